# Initial kernel scaffold; baseline (speedup 1.0000x reference)
import jax
import jax.numpy as jnp
from jax.experimental import pallas as pl


def kernel(x, adj, w_0, b_0, w_1, b_1, w_2, b_2):
    raise NotImplementedError("write your pallas kernel here")


# speedup vs baseline: 1.0239x; 1.0239x over previous
import functools

import jax
import jax.numpy as jnp
from jax.experimental import pallas as pl
from jax.experimental.pallas import tpu as pltpu

_VMEM_LIMIT = 48 * 1024 * 1024


def _prep_kernel(adj_ref, x_ref, w0_ref, abf_ref, d_ref, u0_ref):
    a = adj_ref[...]
    abf_ref[...] = a.astype(jnp.bfloat16)
    d = jax.lax.rsqrt(jnp.sum(a, axis=1, keepdims=True) + 1.0)
    d_ref[...] = d
    z = jnp.dot(x_ref[...].astype(jnp.bfloat16), w0_ref[...],
                preferred_element_type=jnp.float32)
    u0_ref[...] = (d * z).astype(jnp.bfloat16)


def _mid_kernel(abf_ref, u_ref, d_ref, b_ref, w_ref, un_ref, *, row_tile):
    m = pl.program_id(0)
    r0 = pl.multiple_of(m * row_tile, row_tile)
    agg = jnp.dot(abf_ref[...], u_ref[...],
                  preferred_element_type=jnp.float32)
    u_strip = u_ref[pl.ds(r0, row_tile), :].astype(jnp.float32)
    d = d_ref[...]
    out = d * (agg + u_strip) + b_ref[...]
    h = jnp.maximum(out, 0.0).astype(jnp.bfloat16)
    z = jnp.dot(h, w_ref[...], preferred_element_type=jnp.float32)
    un_ref[...] = (d * z).astype(jnp.bfloat16)


def _final_kernel(abf_ref, u_ref, d_ref, b_ref, o_ref, *, row_tile):
    m = pl.program_id(0)
    r0 = pl.multiple_of(m * row_tile, row_tile)
    agg = jnp.dot(abf_ref[...], u_ref[...],
                  preferred_element_type=jnp.float32)
    u_strip = u_ref[pl.ds(r0, row_tile), :].astype(jnp.float32)
    o_ref[...] = d_ref[...] * (agg + u_strip) + b_ref[...]


def kernel(x, adj, w_0, b_0, w_1, b_1, w_2, b_2):
    n, f_in = x.shape
    f_h = w_1.shape[0]
    f_out = w_2.shape[1]
    row_tile = 256
    nstrips = n // row_tile

    w0 = w_0.astype(jnp.bfloat16)
    w1 = w_1.astype(jnp.bfloat16)
    w2 = w_2.astype(jnp.bfloat16)
    b0 = b_0.reshape(1, -1).astype(jnp.float32)
    b1 = b_1.reshape(1, -1).astype(jnp.float32)
    b2 = b_2.reshape(1, -1).astype(jnp.float32)

    cparams = pltpu.CompilerParams(
        dimension_semantics=("parallel",),
        vmem_limit_bytes=_VMEM_LIMIT,
    )
    strip = lambda m: (m, 0)
    const = lambda m: (0, 0)

    abf, d_is, u0 = pl.pallas_call(
        _prep_kernel,
        grid=(nstrips,),
        in_specs=[
            pl.BlockSpec((row_tile, n), strip),
            pl.BlockSpec((row_tile, f_in), strip),
            pl.BlockSpec((f_in, f_h), const),
        ],
        out_specs=[
            pl.BlockSpec((row_tile, n), strip),
            pl.BlockSpec((row_tile, 1), strip),
            pl.BlockSpec((row_tile, f_h), strip),
        ],
        out_shape=[
            jax.ShapeDtypeStruct((n, n), jnp.bfloat16),
            jax.ShapeDtypeStruct((n, 1), jnp.float32),
            jax.ShapeDtypeStruct((n, f_h), jnp.bfloat16),
        ],
        compiler_params=cparams,
    )(adj, x, w0)

    def mid(u, b, w):
        f_cur = u.shape[1]
        f_next = w.shape[1]
        return pl.pallas_call(
            functools.partial(_mid_kernel, row_tile=row_tile),
            grid=(nstrips,),
            in_specs=[
                pl.BlockSpec((row_tile, n), strip),
                pl.BlockSpec((n, f_cur), const),
                pl.BlockSpec((row_tile, 1), strip),
                pl.BlockSpec((1, f_cur), const),
                pl.BlockSpec((f_cur, f_next), const),
            ],
            out_specs=pl.BlockSpec((row_tile, f_next), strip),
            out_shape=jax.ShapeDtypeStruct((n, f_next), jnp.bfloat16),
            compiler_params=cparams,
        )(abf, u, d_is, b, w)

    u1 = mid(u0, b0, w1)
    u2 = mid(u1, b1, w2)

    out = pl.pallas_call(
        functools.partial(_final_kernel, row_tile=row_tile),
        grid=(nstrips,),
        in_specs=[
            pl.BlockSpec((row_tile, n), strip),
            pl.BlockSpec((n, f_out), const),
            pl.BlockSpec((row_tile, 1), strip),
            pl.BlockSpec((1, f_out), const),
        ],
        out_specs=pl.BlockSpec((row_tile, f_out), strip),
        out_shape=jax.ShapeDtypeStruct((n, f_out), jnp.float32),
        compiler_params=cparams,
    )(abf, u2, d_is, b2)

    return out
